# baseline (speedup 1.0000x reference)
import functools

import jax
import jax.numpy as jnp
from jax import lax
from jax.experimental import pallas as pl
from jax.experimental.pallas import tpu as pltpu
from jax.experimental.pallas import tpu_sc as plsc

N = 10000
E = 160000
D = 256
H = 128
NC = 2
NS = 16
CHK = 128

EPT = E // NS
EPT_PAD = 10240
GCHK = 2 * CHK
NCHUNK = EPT_PAD // CHK
EPW = E // (NC * NS)
EPW_PAD = 5120
NCHUNK_D = EPW_PAD // CHK
NPAD = 10240
DUMP = 10048
RPT = NPAD // NS


@functools.cache
def _sc_kernels():
    mesh = plsc.VectorSubcoreMesh(core_axis_name="c", subcore_axis_name="s",
                                  num_cores=NC, num_subcores=NS)

    scatter_kernel = functools.partial(
        pl.kernel,
        out_type=(jax.ShapeDtypeStruct((NPAD, H), jnp.float32),
                  jax.ShapeDtypeStruct((NPAD, H), jnp.float32)),
        mesh=mesh,
        scratch_types=[
            pltpu.VMEM((EPT_PAD // 2,), jnp.int32),
            pltpu.VMEM((NCHUNK // 2, CHK), jnp.int32),
            pltpu.VMEM((GCHK, H), jnp.float32),
            pltpu.VMEM_SHARED((NPAD, H), jnp.float32),
            pltpu.SemaphoreType.DMA,
        ],
    )(_scatter_body)

    deg_kernel = functools.partial(
        pl.kernel,
        out_type=(jax.ShapeDtypeStruct((NPAD, H), jnp.float32),
                  jax.ShapeDtypeStruct((NPAD, H), jnp.float32)),
        mesh=mesh,
        scratch_types=[
            pltpu.VMEM((NCHUNK_D, CHK), jnp.int32),
            pltpu.VMEM((CHK, H), jnp.float32),
            pltpu.VMEM_SHARED((NPAD, H), jnp.float32),
        ],
    )(_deg_body)
    return scatter_kernel, deg_kernel


def _scatter_body(y0_hbm, y1_hbm, srcp_hbm, dstp_hbm, z0_hbm, z1_hbm,
                  src_v, dst_v, rows_v, z_sh, sem0):
    c = lax.axis_index("c")
    s = lax.axis_index("s")

    def run(y_hbm, z_hbm):
        pltpu.sync_copy(y_hbm.at[pl.ds(s * RPT, RPT)],
                        z_sh.at[pl.ds(s * RPT, RPT)])
        plsc.subcore_barrier()

        half = NCHUNK // 2
        ehalf = EPT_PAD // 2

        def outer(p, carry):
            pltpu.sync_copy(srcp_hbm.at[s, pl.ds(p * ehalf, ehalf)], src_v)
            pltpu.sync_copy(dstp_hbm.at[s, pl.ds(p * half, half)], dst_v)

            def body(j, carry2):
                h = pltpu.async_copy(
                    y_hbm.at[src_v.at[pl.ds(j * GCHK, GCHK)]], rows_v, sem0)
                h.wait()
                pltpu.sync_copy(rows_v.at[pl.ds(0, CHK)],
                                z_sh.at[dst_v.at[2 * j]], add=True)
                pltpu.sync_copy(rows_v.at[pl.ds(CHK, CHK)],
                                z_sh.at[dst_v.at[2 * j + 1]], add=True)
                return carry2

            lax.fori_loop(0, half // 2, body, 0)
            return carry

        lax.fori_loop(0, 2, outer, 0)
        plsc.subcore_barrier()
        pltpu.sync_copy(z_sh.at[pl.ds(s * RPT, RPT)],
                        z_hbm.at[pl.ds(s * RPT, RPT)])

    @pl.when(c == 0)
    def _():
        run(y0_hbm, z0_hbm)

    @pl.when(c == 1)
    def _():
        run(y1_hbm, z1_hbm)


def _deg_body(ones_hbm, zeros_hbm, dst0_hbm, dst1_hbm, p0_hbm, p1_hbm,
              dst_v, ones_v, t_sh):
    c = lax.axis_index("c")
    s = lax.axis_index("s")
    pltpu.sync_copy(ones_hbm, ones_v)
    pltpu.sync_copy(zeros_hbm, t_sh.at[pl.ds(s * RPT, RPT)])

    def run(dst_hbm, p_hbm):
        pltpu.sync_copy(dst_hbm.at[s], dst_v)
        plsc.subcore_barrier()

        def body(j, carry):
            pltpu.sync_copy(ones_v, t_sh.at[dst_v.at[j]], add=True)
            return carry

        lax.fori_loop(0, NCHUNK_D, body, 0)
        plsc.subcore_barrier()
        pltpu.sync_copy(t_sh.at[pl.ds(s * RPT, RPT)],
                        p_hbm.at[pl.ds(s * RPT, RPT)])

    @pl.when(c == 0)
    def _():
        run(dst0_hbm, p0_hbm)

    @pl.when(c == 1)
    def _():
        run(dst1_hbm, p1_hbm)


_R = 1000


def _dinv_block(d0_ref, d1_ref):
    return lax.rsqrt(d0_ref[:, 0:1] + d1_ref[:, 0:1] + 1.0)


def _tc_first_body(x_ref, w_ref, d0_ref, d1_ref, y0_ref, y1_ref):
    dinv = _dinv_block(d0_ref, d1_ref)
    y = jnp.dot(x_ref[...], w_ref[...],
                preferred_element_type=jnp.float32) * dinv
    y0_ref[...] = y[:, :H]
    y1_ref[...] = y[:, H:]


def _tc_mid_body(z0_ref, z1_ref, w_ref, b_ref, d0_ref, d1_ref,
                 y0_ref, y1_ref):
    dinv = _dinv_block(d0_ref, d1_ref)
    z = jnp.concatenate([z0_ref[...], z1_ref[...]], axis=1)
    h = jnp.maximum(z * dinv + b_ref[...], 0.0)
    y = jnp.dot(h, w_ref[...], preferred_element_type=jnp.float32) * dinv
    y0_ref[...] = y[:, :H]
    y1_ref[...] = y[:, H:]


def _tc_last_body(z0_ref, z1_ref, b_ref, d0_ref, d1_ref, out_ref):
    dinv = _dinv_block(d0_ref, d1_ref)
    z = jnp.concatenate([z0_ref[...], z1_ref[...]], axis=1)
    out_ref[...] = z * dinv + b_ref[...]


_half_spec = pl.BlockSpec((_R, H), lambda i: (i, 0))
_full_spec = pl.BlockSpec((_R, D), lambda i: (i, 0))
_w_spec = pl.BlockSpec((D, D), lambda i: (0, 0))
_b_spec = pl.BlockSpec((1, D), lambda i: (0, 0))
_deg_spec = pl.BlockSpec((_R, H), lambda i: (i, 0))
_GRID = (N // _R,)

_y_shape = (jax.ShapeDtypeStruct((NPAD, H), jnp.float32),
            jax.ShapeDtypeStruct((NPAD, H), jnp.float32))

_tc_first = pl.pallas_call(
    _tc_first_body,
    grid=_GRID,
    in_specs=[_full_spec, _w_spec, _deg_spec, _deg_spec],
    out_specs=[_half_spec, _half_spec],
    out_shape=_y_shape,
)

_tc_mid = pl.pallas_call(
    _tc_mid_body,
    grid=_GRID,
    in_specs=[_half_spec, _half_spec, _w_spec, _b_spec, _deg_spec, _deg_spec],
    out_specs=[_half_spec, _half_spec],
    out_shape=_y_shape,
)

_tc_last = pl.pallas_call(
    _tc_last_body,
    grid=_GRID,
    in_specs=[_half_spec, _half_spec, _b_spec, _deg_spec, _deg_spec],
    out_specs=_full_spec,
    out_shape=jax.ShapeDtypeStruct((N, D), jnp.float32),
)


def kernel(x, edge_index, W1, b1, W2, b2, W3, b3):
    src = edge_index[0]
    dst = edge_index[1]
    srcp = jnp.pad(src.reshape(NS, EPT), ((0, 0), (0, EPT_PAD - EPT)))
    dstp = jnp.pad(dst.reshape(NS, EPT), ((0, 0), (0, EPT_PAD - EPT)),
                   constant_values=DUMP).reshape(NS, NCHUNK, CHK)
    dst_halves = jnp.pad(dst.reshape(NC * NS, EPW),
                         ((0, 0), (0, EPW_PAD - EPW)),
                         constant_values=DUMP)
    dst0 = dst_halves[:NS].reshape(NS, NCHUNK_D, CHK)
    dst1 = dst_halves[NS:].reshape(NS, NCHUNK_D, CHK)
    ones_rows = jnp.ones((CHK, H), jnp.float32)
    zero_rows = jnp.zeros((RPT, H), jnp.float32)

    _scatter_kernel, _deg_kernel = _sc_kernels()
    p0, p1 = _deg_kernel(ones_rows, zero_rows, dst0, dst1)

    y0, y1 = _tc_first(x, W1, p0, p1)
    z0, z1 = _scatter_kernel(y0, y1, srcp, dstp)

    y0, y1 = _tc_mid(z0, z1, W2, b1.reshape(1, D), p0, p1)
    z0, z1 = _scatter_kernel(y0, y1, srcp, dstp)

    y0, y1 = _tc_mid(z0, z1, W3, b2.reshape(1, D), p0, p1)
    z0, z1 = _scatter_kernel(y0, y1, srcp, dstp)

    return _tc_last(z0, z1, b3.reshape(1, D), p0, p1)


# speedup vs baseline: 1.0046x; 1.0046x over previous
import functools

import jax
import jax.numpy as jnp
from jax import lax
from jax.experimental import pallas as pl
from jax.experimental.pallas import tpu as pltpu
from jax.experimental.pallas import tpu_sc as plsc

N = 10000
E = 160000
D = 256
H = 128
NC = 2
NS = 16
CHK = 128

EPT = E // NS
EPT_PAD = 10240
GCHK = 2 * CHK
NCHUNK = EPT_PAD // GCHK
EPW = E // (NC * NS)
EPW_PAD = 5120
NCHUNK_D = EPW_PAD // CHK
NPAD = 10240
DUMP = 10048
RPT = NPAD // NS


@functools.cache
def _sc_kernels():
    mesh = plsc.VectorSubcoreMesh(core_axis_name="c", subcore_axis_name="s",
                                  num_cores=NC, num_subcores=NS)

    scatter_kernel = functools.partial(
        pl.kernel,
        out_type=(jax.ShapeDtypeStruct((NPAD, H), jnp.float32),
                  jax.ShapeDtypeStruct((NPAD, H), jnp.float32)),
        mesh=mesh,
        scratch_types=[
            pltpu.VMEM((EPT_PAD // 2,), jnp.int32),
            pltpu.VMEM((EPT_PAD,), jnp.int32),
            pltpu.VMEM((GCHK, H), jnp.float32),
            pltpu.VMEM_SHARED((NPAD, H), jnp.float32),
            pltpu.SemaphoreType.DMA,
        ],
    )(_scatter_body)

    deg_kernel = functools.partial(
        pl.kernel,
        out_type=(jax.ShapeDtypeStruct((NPAD, H), jnp.float32),
                  jax.ShapeDtypeStruct((NPAD, H), jnp.float32)),
        mesh=mesh,
        scratch_types=[
            pltpu.VMEM((NCHUNK_D, CHK), jnp.int32),
            pltpu.VMEM((CHK, H), jnp.float32),
            pltpu.VMEM_SHARED((NPAD, H), jnp.float32),
        ],
    )(_deg_body)
    return scatter_kernel, deg_kernel


def _scatter_body(y0_hbm, y1_hbm, srcp_hbm, dstp_hbm, z0_hbm, z1_hbm,
                  src_v, dst_v, rows_v, z_sh, sem0):
    c = lax.axis_index("c")
    s = lax.axis_index("s")

    def run(y_hbm, z_hbm):
        pltpu.sync_copy(y_hbm.at[pl.ds(s * RPT, RPT)],
                        z_sh.at[pl.ds(s * RPT, RPT)])
        plsc.subcore_barrier()

        half = NCHUNK // 2
        ehalf = EPT_PAD // 2
        pltpu.sync_copy(dstp_hbm.at[s], dst_v)

        def outer(p, carry):
            pltpu.sync_copy(srcp_hbm.at[s, pl.ds(p * ehalf, ehalf)], src_v)

            def body(j, carry2):
                h = pltpu.async_copy(
                    y_hbm.at[src_v.at[pl.ds(j * GCHK, GCHK)]], rows_v, sem0)
                h.wait()
                pltpu.sync_copy(
                    rows_v,
                    z_sh.at[dst_v.at[pl.ds((p * half + j) * GCHK, GCHK)]],
                    add=True)
                return carry2

            lax.fori_loop(0, half, body, 0)
            return carry

        lax.fori_loop(0, 2, outer, 0)
        plsc.subcore_barrier()
        pltpu.sync_copy(z_sh.at[pl.ds(s * RPT, RPT)],
                        z_hbm.at[pl.ds(s * RPT, RPT)])

    @pl.when(c == 0)
    def _():
        run(y0_hbm, z0_hbm)

    @pl.when(c == 1)
    def _():
        run(y1_hbm, z1_hbm)


def _deg_body(ones_hbm, zeros_hbm, dst0_hbm, dst1_hbm, p0_hbm, p1_hbm,
              dst_v, ones_v, t_sh):
    c = lax.axis_index("c")
    s = lax.axis_index("s")
    pltpu.sync_copy(ones_hbm, ones_v)
    pltpu.sync_copy(zeros_hbm, t_sh.at[pl.ds(s * RPT, RPT)])

    def run(dst_hbm, p_hbm):
        pltpu.sync_copy(dst_hbm.at[s], dst_v)
        plsc.subcore_barrier()

        def body(j, carry):
            pltpu.sync_copy(ones_v, t_sh.at[dst_v.at[j]], add=True)
            return carry

        lax.fori_loop(0, NCHUNK_D, body, 0)
        plsc.subcore_barrier()
        pltpu.sync_copy(t_sh.at[pl.ds(s * RPT, RPT)],
                        p_hbm.at[pl.ds(s * RPT, RPT)])

    @pl.when(c == 0)
    def _():
        run(dst0_hbm, p0_hbm)

    @pl.when(c == 1)
    def _():
        run(dst1_hbm, p1_hbm)


_R = 1000


def _dinv_block(d0_ref, d1_ref):
    return lax.rsqrt(d0_ref[:, 0:1] + d1_ref[:, 0:1] + 1.0)


def _tc_first_body(x_ref, w_ref, d0_ref, d1_ref, y0_ref, y1_ref):
    dinv = _dinv_block(d0_ref, d1_ref)
    y = jnp.dot(x_ref[...], w_ref[...],
                preferred_element_type=jnp.float32) * dinv
    y0_ref[...] = y[:, :H]
    y1_ref[...] = y[:, H:]


def _tc_mid_body(z0_ref, z1_ref, w_ref, b_ref, d0_ref, d1_ref,
                 y0_ref, y1_ref):
    dinv = _dinv_block(d0_ref, d1_ref)
    z = jnp.concatenate([z0_ref[...], z1_ref[...]], axis=1)
    h = jnp.maximum(z * dinv + b_ref[...], 0.0)
    y = jnp.dot(h, w_ref[...], preferred_element_type=jnp.float32) * dinv
    y0_ref[...] = y[:, :H]
    y1_ref[...] = y[:, H:]


def _tc_last_body(z0_ref, z1_ref, b_ref, d0_ref, d1_ref, out_ref):
    dinv = _dinv_block(d0_ref, d1_ref)
    z = jnp.concatenate([z0_ref[...], z1_ref[...]], axis=1)
    out_ref[...] = z * dinv + b_ref[...]


_half_spec = pl.BlockSpec((_R, H), lambda i: (i, 0))
_full_spec = pl.BlockSpec((_R, D), lambda i: (i, 0))
_w_spec = pl.BlockSpec((D, D), lambda i: (0, 0))
_b_spec = pl.BlockSpec((1, D), lambda i: (0, 0))
_deg_spec = pl.BlockSpec((_R, H), lambda i: (i, 0))
_GRID = (N // _R,)

_y_shape = (jax.ShapeDtypeStruct((NPAD, H), jnp.float32),
            jax.ShapeDtypeStruct((NPAD, H), jnp.float32))

_tc_first = pl.pallas_call(
    _tc_first_body,
    grid=_GRID,
    in_specs=[_full_spec, _w_spec, _deg_spec, _deg_spec],
    out_specs=[_half_spec, _half_spec],
    out_shape=_y_shape,
)

_tc_mid = pl.pallas_call(
    _tc_mid_body,
    grid=_GRID,
    in_specs=[_half_spec, _half_spec, _w_spec, _b_spec, _deg_spec, _deg_spec],
    out_specs=[_half_spec, _half_spec],
    out_shape=_y_shape,
)

_tc_last = pl.pallas_call(
    _tc_last_body,
    grid=_GRID,
    in_specs=[_half_spec, _half_spec, _b_spec, _deg_spec, _deg_spec],
    out_specs=_full_spec,
    out_shape=jax.ShapeDtypeStruct((N, D), jnp.float32),
)


def kernel(x, edge_index, W1, b1, W2, b2, W3, b3):
    src = edge_index[0]
    dst = edge_index[1]
    srcp = jnp.pad(src.reshape(NS, EPT), ((0, 0), (0, EPT_PAD - EPT)))
    dstp = jnp.pad(dst.reshape(NS, EPT), ((0, 0), (0, EPT_PAD - EPT)),
                   constant_values=DUMP)
    dst_halves = jnp.pad(dst.reshape(NC * NS, EPW),
                         ((0, 0), (0, EPW_PAD - EPW)),
                         constant_values=DUMP)
    dst0 = dst_halves[:NS].reshape(NS, NCHUNK_D, CHK)
    dst1 = dst_halves[NS:].reshape(NS, NCHUNK_D, CHK)
    ones_rows = jnp.ones((CHK, H), jnp.float32)
    zero_rows = jnp.zeros((RPT, H), jnp.float32)

    _scatter_kernel, _deg_kernel = _sc_kernels()
    p0, p1 = _deg_kernel(ones_rows, zero_rows, dst0, dst1)

    y0, y1 = _tc_first(x, W1, p0, p1)
    z0, z1 = _scatter_kernel(y0, y1, srcp, dstp)

    y0, y1 = _tc_mid(z0, z1, W2, b1.reshape(1, D), p0, p1)
    z0, z1 = _scatter_kernel(y0, y1, srcp, dstp)

    y0, y1 = _tc_mid(z0, z1, W3, b2.reshape(1, D), p0, p1)
    z0, z1 = _scatter_kernel(y0, y1, srcp, dstp)

    return _tc_last(z0, z1, b3.reshape(1, D), p0, p1)
